# baseline (speedup 1.0000x reference)
import functools

import jax
import jax.numpy as jnp
from jax import lax
from jax.experimental import pallas as pl
from jax.experimental.pallas import tpu as pltpu
from jax.experimental.pallas import tpu_sc as plsc

B = 16384
C = 1000
K = 200
NC = 2
NS = 16
NW = NC * NS
K_ST = 136
K_SP = K - K_ST
ST_BASE = K_ST // NW
ST_REM = K_ST % NW
MAX_ST = ST_BASE + 1
SP_PER_W = K_SP // NW

_mesh = plsc.VectorSubcoreMesh(
    core_axis_name="c", subcore_axis_name="s", num_cores=NC, num_subcores=NS
)


@functools.partial(
    pl.kernel,
    out_type=jax.ShapeDtypeStruct((K, B), jnp.float32),
    mesh=_mesh,
    scratch_types=[
        pltpu.VMEM((K + 24,), jnp.int32),
        *[pltpu.VMEM((B,), jnp.float32) for _ in range(MAX_ST)],
        pltpu.VMEM_SHARED((NS * SP_PER_W, B), jnp.float32),
        *[pltpu.SemaphoreType.DMA for _ in range(MAX_ST)],
        *[pltpu.SemaphoreType.DMA for _ in range(SP_PER_W)],
        pltpu.SemaphoreType.DMA,
    ],
    compiler_params=pltpu.CompilerParams(needs_layout_passes=False),
)
def _row_gather(xt_hbm, mask_hbm, out_hbm, mask_v, *rest):
    rows = rest[:MAX_ST]
    spmem = rest[MAX_ST]
    sem_in = rest[MAX_ST + 1:2 * MAX_ST + 1]
    sem_sp = rest[2 * MAX_ST + 1:2 * MAX_ST + 1 + SP_PER_W]
    sem_out = rest[2 * MAX_ST + 1 + SP_PER_W]

    cid = lax.axis_index("c")
    sid = lax.axis_index("s")
    wid = sid * NC + cid

    pltpu.sync_copy(mask_hbm, mask_v.at[pl.ds(0, K)])
    lane0 = lax.iota(jnp.int32, 16) == 0

    def src_row(j):
        mv = mask_v[pl.ds(j, 16)]
        return jnp.sum(jnp.where(lane0, mv, 0))

    st_cnt = jnp.where(wid < ST_REM, ST_BASE + 1, ST_BASE)
    st_start = wid * ST_BASE + jnp.minimum(wid, ST_REM)
    sp_start = K_ST + cid * NS * SP_PER_W + SP_PER_W * sid

    def g_desc(j, i):
        return pltpu.make_async_copy(xt_hbm.at[src_row(j)], rows[i], sem_in[i])

    def p_desc(j, i):
        return pltpu.make_async_copy(rows[i], out_hbm.at[j], sem_out)

    def gsp_desc(j, i):
        slot = SP_PER_W * sid + i
        return pltpu.make_async_copy(
            xt_hbm.at[src_row(j)], spmem.at[slot], sem_sp[i]
        )

    def psp_desc(j, i):
        slot = SP_PER_W * sid + i
        return pltpu.make_async_copy(spmem.at[slot], out_hbm.at[j], sem_out)

    for i in range(SP_PER_W):
        gsp_desc(sp_start + i, i).start()
    for i in range(MAX_ST):
        @pl.when(i < st_cnt)
        def _():
            g_desc(st_start + i, i).start()

    for i in range(MAX_ST):
        @pl.when(i < st_cnt)
        def _():
            g_desc(st_start + i, i).wait()
            p_desc(st_start + i, i).start()
    for i in range(SP_PER_W):
        gsp_desc(sp_start + i, i).wait()
        psp_desc(sp_start + i, i).start()

    for i in range(MAX_ST):
        @pl.when(i < st_cnt)
        def _():
            p_desc(st_start + i, i).wait()
    for i in range(SP_PER_W):
        psp_desc(sp_start + i, i).wait()


def kernel(x, mask):
    return _row_gather(x.T, mask).T
